# baseline (speedup 1.0000x reference)
import functools

import jax
import jax.numpy as jnp
from jax import lax
from jax.experimental import pallas as pl
from jax.experimental.pallas import tpu as pltpu
from jax.experimental.pallas import tpu_sc as plsc

VOCAB = 104353
DIM = 128
B = 4096
L = 50

NC = 2
NS = 16
NW = NC * NS

PER_B = B // NW
GB = 2
GL = GB * L
PER_G = PER_B // GB
NBUF = 8
NG = PER_G // NBUF


def _make_gather():
    mesh = plsc.VectorSubcoreMesh(core_axis_name="c", subcore_axis_name="s")

    @functools.partial(
        pl.kernel,
        mesh=mesh,
        out_type=jax.ShapeDtypeStruct((B // GB, GL, DIM), jnp.float32),
        scratch_types=[
            pltpu.VMEM((PER_G, GL), jnp.int32),
            *[pltpu.VMEM((1, GL, DIM), jnp.float32) for _ in range(NBUF)],
            pltpu.SemaphoreType.DMA((NBUF,)),
            pltpu.SemaphoreType.DMA((NBUF,)),
        ],
    )
    def gather_kernel(x_hbm, table_hbm, out_hbm, idx_v, *bufs_and_sems):
        bufs = bufs_and_sems[:NBUF]
        gsem, wsem = bufs_and_sems[NBUF], bufs_and_sems[NBUF + 1]
        wid = lax.axis_index("s") * NC + lax.axis_index("c")
        base = wid * PER_G
        pltpu.sync_copy(x_hbm.at[wid], idx_v)

        def gather_start(j, r):
            pltpu.make_async_copy(
                table_hbm.at[idx_v.at[j]], bufs[r].at[0], gsem.at[r]
            ).start()

        def gather_wait(j, r):
            pltpu.make_async_copy(
                table_hbm.at[idx_v.at[j]], bufs[r].at[0], gsem.at[r]
            ).wait()

        def wb_start(j, r):
            pltpu.make_async_copy(
                bufs[r], out_hbm.at[pl.ds(base + j, 1)], wsem.at[r]
            ).start()

        def wb_wait(j, r):
            pltpu.make_async_copy(
                bufs[r], out_hbm.at[pl.ds(base + j, 1)], wsem.at[r]
            ).wait()

        for r in range(NBUF):
            gather_start(r, r)

        def body(g, carry):
            for r in range(NBUF):
                j = g * NBUF + r
                gather_wait(j, r)
                wb_start(j, r)
            for r in range(NBUF):
                j = g * NBUF + r
                wb_wait(j, r)
                gather_start(j + NBUF, r)
            return carry

        lax.fori_loop(0, NG - 1, body, 0)

        for r in range(NBUF):
            j = (NG - 1) * NBUF + r
            gather_wait(j, r)
            wb_start(j, r)
        for r in range(NBUF):
            j = (NG - 1) * NBUF + r
            wb_wait(j, r)

    return gather_kernel


_gather = _make_gather()


@jax.jit
def kernel(x, table):
    xf = x.reshape(NW, PER_G, GL)
    return _gather(xf, table).reshape(B, L, DIM)


# speedup vs baseline: 1.7962x; 1.7962x over previous
import functools

import jax
import jax.numpy as jnp
from jax import lax
from jax.experimental import pallas as pl
from jax.experimental.pallas import tpu as pltpu
from jax.experimental.pallas import tpu_sc as plsc

VOCAB = 104353
DIM = 128
B = 4096
L = 50

NC = 2
NS = 16
NW = NC * NS

PER_B = B // NW
GB = 1
GL = GB * L
PER_G = PER_B // GB
NBUF = 8
NG = PER_G // NBUF


def _make_gather():
    mesh = plsc.VectorSubcoreMesh(core_axis_name="c", subcore_axis_name="s")

    @functools.partial(
        pl.kernel,
        mesh=mesh,
        out_type=jax.ShapeDtypeStruct((B // GB, GL, DIM), jnp.float32),
        scratch_types=[
            pltpu.VMEM((PER_G, GL), jnp.int32),
            *[pltpu.VMEM((1, GL, DIM), jnp.float32) for _ in range(NBUF)],
            pltpu.SemaphoreType.DMA((NBUF,)),
            pltpu.SemaphoreType.DMA((NBUF,)),
        ],
    )
    def gather_kernel(x_hbm, table_hbm, out_hbm, idx_v, *bufs_and_sems):
        bufs = bufs_and_sems[:NBUF]
        gsem, wsem = bufs_and_sems[NBUF], bufs_and_sems[NBUF + 1]
        wid = lax.axis_index("s") * NC + lax.axis_index("c")
        base = wid * PER_G
        pltpu.sync_copy(x_hbm.at[wid], idx_v)

        def gather_start(j, r):
            pltpu.make_async_copy(
                table_hbm.at[idx_v.at[j]], bufs[r].at[0], gsem.at[r]
            ).start()

        def gather_wait(j, r):
            pltpu.make_async_copy(
                table_hbm.at[idx_v.at[j]], bufs[r].at[0], gsem.at[r]
            ).wait()

        def wb_start(j, r):
            pltpu.make_async_copy(
                bufs[r], out_hbm.at[pl.ds(base + j, 1)], wsem.at[r]
            ).start()

        def wb_wait(j, r):
            pltpu.make_async_copy(
                bufs[r], out_hbm.at[pl.ds(base + j, 1)], wsem.at[r]
            ).wait()

        for r in range(NBUF):
            gather_start(r, r)

        def body(g, carry):
            for r in range(NBUF):
                j = g * NBUF + r
                gather_wait(j, r)
                wb_start(j, r)
            for r in range(NBUF):
                j = g * NBUF + r
                wb_wait(j, r)
                gather_start(j + NBUF, r)
            return carry

        lax.fori_loop(0, NG - 1, body, 0)

        for r in range(NBUF):
            j = (NG - 1) * NBUF + r
            gather_wait(j, r)
            wb_start(j, r)
        for r in range(NBUF):
            j = (NG - 1) * NBUF + r
            wb_wait(j, r)

    return gather_kernel


_gather = _make_gather()


@jax.jit
def kernel(x, table):
    xf = x.reshape(NW, PER_G, GL)
    return _gather(xf, table).reshape(B, L, DIM)
